# baseline (speedup 1.0000x reference)
import functools

import jax
import jax.numpy as jnp
from jax import lax
from jax.experimental import pallas as pl
from jax.experimental.pallas import tpu as pltpu
from jax.experimental.pallas import tpu_sc as plsc


def _sc_gather(table, idx, n, d):
    nw = 32
    b_per_w = n // nw
    unroll = 8
    mesh = plsc.VectorSubcoreMesh(core_axis_name="c", subcore_axis_name="s")

    @functools.partial(
        pl.kernel,
        mesh=mesh,
        compiler_params=pltpu.CompilerParams(needs_layout_passes=False),
        out_type=jax.ShapeDtypeStruct((n, d), jnp.float32),
        scratch_types=[
            pltpu.VMEM((b_per_w,), jnp.int32),
            pltpu.SemaphoreType.DMA,
        ],
    )
    def k(table_hbm, idx_hbm, out_hbm, idx_v, sem):
        wid = lax.axis_index("s") * mesh.num_cores + lax.axis_index("c")
        base = wid * b_per_w
        pltpu.sync_copy(idx_hbm.at[pl.ds(base, b_per_w)], idx_v)
        lanes = lax.iota(jnp.int32, 16)

        def body(g, carry):
            vec = idx_v[pl.ds(g * 16, 16)]
            for l in range(16):
                i = jnp.sum(jnp.where(lanes == l, vec, 0))
                pltpu.async_copy(
                    table_hbm.at[pl.ds(i, 1)],
                    out_hbm.at[pl.ds(base + g * 16 + l, 1)],
                    sem,
                )
            return carry

        lax.fori_loop(0, b_per_w // 16, body, 0)
        pltpu.make_async_copy(
            table_hbm.at[pl.ds(0, b_per_w)],
            out_hbm.at[pl.ds(base, b_per_w)],
            sem,
        ).wait()

    return k(table, idx)


def _tc_body(mus_ref, r_ref, ann_ref, conf_ref, out_ref):
    mu = mus_ref[...]
    r = r_ref[...]
    a = ann_ref[...]
    cf = conf_ref[...]
    blk, dd = r.shape
    iota = lax.broadcasted_iota(jnp.int32, (blk, dd), 1)
    onehot = (iota == a).astype(jnp.float32)
    r_pick = jnp.sum(r * onehot, axis=1, keepdims=True)
    dn = (((1,), (1,)), ((), ()))
    mu_pick = lax.dot_general(onehot, mu, dn,
                              preferred_element_type=jnp.float32)
    p = lax.dot_general(jnp.exp(r), jnp.exp(mu), dn,
                        preferred_element_type=jnp.float32)
    out_ref[...] = cf * (mu_pick + r_pick - jnp.log(p))


def _tc_compute(r, mus, ann2, conf2, blk):
    n, d = r.shape
    c = mus.shape[0]
    grid = n // blk
    return pl.pallas_call(
        _tc_body,
        grid=(grid,),
        in_specs=[
            pl.BlockSpec((c, d), lambda i: (0, 0)),
            pl.BlockSpec((blk, d), lambda i: (i, 0)),
            pl.BlockSpec((blk, 1), lambda i: (i, 0)),
            pl.BlockSpec((blk, 1), lambda i: (i, 0)),
        ],
        out_specs=pl.BlockSpec((blk, c), lambda i: (i, 0)),
        out_shape=jax.ShapeDtypeStruct((n, c), jnp.float32),
    )(mus, r, ann2, conf2)


def kernel(mus, random_effects, annotators, annotations, confidences):
    n = annotators.shape[0]
    d = random_effects.shape[1]
    r = _sc_gather(random_effects, annotators, n, d)
    ann2 = annotations.reshape(n, 1)
    conf2 = confidences.reshape(n, 1)
    return _tc_compute(r, mus, ann2, conf2, blk=2048)


# speedup vs baseline: 3.0877x; 3.0877x over previous
import functools

import jax
import jax.numpy as jnp
from jax import lax
from jax.experimental import pallas as pl
from jax.experimental.pallas import tpu as pltpu
from jax.experimental.pallas import tpu_sc as plsc


def _sc_gather(table, idx, n, d):
    nw = 32
    b_per_w = n // nw
    ch = 128
    nch = b_per_w // ch
    mesh = plsc.VectorSubcoreMesh(core_axis_name="c", subcore_axis_name="s")

    @functools.partial(
        pl.kernel,
        mesh=mesh,
        compiler_params=pltpu.CompilerParams(use_tc_tiling_on_sc=False),
        out_type=jax.ShapeDtypeStruct((n, d), jnp.float32),
        scratch_types=[
            pltpu.VMEM((b_per_w,), jnp.int32),
            pltpu.VMEM((b_per_w, d), jnp.float32),
            pltpu.SemaphoreType.DMA,
        ],
    )
    def k(table_hbm, idx_hbm, out_hbm, idx_v, rows_v, sem):
        wid = lax.axis_index("s") * mesh.num_cores + lax.axis_index("c")
        base = wid * b_per_w
        pltpu.sync_copy(idx_hbm.at[pl.ds(base, b_per_w)], idx_v)
        copies = [
            pltpu.async_copy(
                table_hbm.at[idx_v.at[pl.ds(j * ch, ch)]],
                rows_v.at[pl.ds(j * ch, ch)],
                sem,
            )
            for j in range(nch)
        ]
        for c in copies:
            c.wait()
        pltpu.sync_copy(rows_v, out_hbm.at[pl.ds(base, b_per_w)])

    return k(table, idx)


def _tc_body(mus_ref, rf_ref, ann_ref, conf_ref, out_ref):
    mu = mus_ref[...]
    rf = rf_ref[...]
    a4 = ann_ref[...]
    c4 = conf_ref[...]
    b4 = rf.shape[0]
    em = jnp.exp(mu)
    jj = lax.broadcasted_iota(jnp.int32, (128, 32), 0)
    kk = lax.broadcasted_iota(jnp.int32, (128, 32), 1)
    gmask = (jj // 32 == kk // 8).astype(jnp.float32)
    m4 = jnp.tile(em.T, (4, 4)) * gmask
    dn = (((1,), (0,)), ((), ()))
    p4 = lax.dot_general(jnp.exp(rf), m4, dn,
                         preferred_element_type=jnp.float32)
    lp4 = jnp.log(p4)
    iota32 = lax.broadcasted_iota(jnp.int32, (b4, 32), 1)
    dn_t = (((1,), (1,)), ((), ()))
    pieces = []
    for g in range(4):
        ag = a4[:, g:g + 1]
        ohg = (iota32 == ag).astype(jnp.float32)
        rg = rf[:, 32 * g:32 * (g + 1)]
        rp = jnp.sum(rg * ohg, axis=1, keepdims=True)
        mp = lax.dot_general(ohg, mu, dn_t,
                             preferred_element_type=jnp.float32)
        pieces.append(c4[:, g:g + 1] * (mp + rp - lp4[:, 8 * g:8 * (g + 1)]))
    out_ref[...] = jnp.concatenate(pieces, axis=1)


def _tc_compute(rf, mus, ann4, conf4, b4):
    n4 = rf.shape[0]
    c, d = mus.shape
    grid = n4 // b4
    return pl.pallas_call(
        _tc_body,
        grid=(grid,),
        in_specs=[
            pl.BlockSpec((c, d), lambda i: (0, 0)),
            pl.BlockSpec((b4, 128), lambda i: (i, 0)),
            pl.BlockSpec((b4, 4), lambda i: (i, 0)),
            pl.BlockSpec((b4, 4), lambda i: (i, 0)),
        ],
        out_specs=pl.BlockSpec((b4, 4 * c), lambda i: (i, 0)),
        out_shape=jax.ShapeDtypeStruct((n4, 4 * c), jnp.float32),
    )(mus, rf, ann4, conf4)


def kernel(mus, random_effects, annotators, annotations, confidences):
    n = annotators.shape[0]
    d = random_effects.shape[1]
    r = _sc_gather(random_effects, annotators, n, d)
    rf = r.reshape(n // 4, 4 * d)
    ann4 = annotations.reshape(n // 4, 4)
    conf4 = confidences.reshape(n // 4, 4)
    packed = _tc_compute(rf, mus, ann4, conf4, b4=512)
    return packed.reshape(n, mus.shape[0])


# speedup vs baseline: 3.4548x; 1.1189x over previous
import functools

import jax
import jax.numpy as jnp
from jax import lax
from jax.experimental import pallas as pl
from jax.experimental.pallas import tpu as pltpu
from jax.experimental.pallas import tpu_sc as plsc


def _sc_gather(table, idx, n, d):
    nw = 32
    b_per_w = n // nw
    ch = 128
    nch = b_per_w // ch
    mesh = plsc.VectorSubcoreMesh(core_axis_name="c", subcore_axis_name="s")

    @functools.partial(
        pl.kernel,
        mesh=mesh,
        compiler_params=pltpu.CompilerParams(use_tc_tiling_on_sc=False),
        out_type=jax.ShapeDtypeStruct((n // 4, 4 * d), jnp.float32),
        scratch_types=[
            pltpu.VMEM((b_per_w,), jnp.int32),
            pltpu.VMEM((b_per_w, d), jnp.float32),
            pltpu.SemaphoreType.DMA,
        ],
    )
    def k(table_hbm, idx_hbm, out_hbm, idx_v, rows_v, sem):
        wid = lax.axis_index("s") * mesh.num_cores + lax.axis_index("c")
        base = wid * b_per_w
        blk = wid // 4
        grp = wid % 4
        pltpu.sync_copy(idx_hbm.at[pl.ds(base, b_per_w)], idx_v)
        copies = [
            pltpu.async_copy(
                table_hbm.at[idx_v.at[pl.ds(j * ch, ch)]],
                rows_v.at[pl.ds(j * ch, ch)],
                sem,
            )
            for j in range(nch)
        ]
        for c in copies:
            c.wait()
        pltpu.sync_copy(
            rows_v,
            out_hbm.at[pl.ds(blk * b_per_w, b_per_w), pl.ds(grp * d, d)],
        )

    return k(table, idx)


def _tc_body(mus_ref, rf_ref, ann_ref, conf_ref, out_ref):
    mu = mus_ref[...]
    rf = rf_ref[...]
    a_col = ann_ref[...]
    c_col = conf_ref[...]
    b = rf.shape[0]
    jj = lax.broadcasted_iota(jnp.int32, (128, 32), 0)
    kk = lax.broadcasted_iota(jnp.int32, (128, 32), 1)
    gmask = (jj // 32 == kk // 8).astype(jnp.float32)
    m4e = jnp.tile(jnp.exp(mu).T, (4, 4)) * gmask
    m4u = jnp.tile(mu.T, (4, 4)) * gmask
    dn = (((1,), (0,)), ((), ()))
    p4 = lax.dot_general(jnp.exp(rf), m4e, dn,
                         preferred_element_type=jnp.float32)
    iota32 = lax.broadcasted_iota(jnp.int32, (b, 32), 1)
    ohs, rps, cfs = [], [], []
    for g in range(4):
        ag = a_col[g * b:(g + 1) * b, :]
        oh = (iota32 == ag).astype(jnp.float32)
        ohs.append(oh)
        rg = rf[:, 32 * g:32 * (g + 1)]
        rps.append(jnp.sum(rg * oh, axis=1, keepdims=True))
        cfs.append(c_col[g * b:(g + 1) * b, :])
    oh128 = jnp.concatenate(ohs, axis=1)
    mp4 = lax.dot_general(oh128, m4u, dn,
                          preferred_element_type=jnp.float32)
    sum4_t = (mp4 - jnp.log(p4)).T
    rp_t = jnp.concatenate(rps, axis=1).T
    cf_t = jnp.concatenate(cfs, axis=1).T
    for g in range(4):
        out_ref[:, g * b:(g + 1) * b] = cf_t[g:g + 1, :] * (
            sum4_t[8 * g:8 * (g + 1), :] + rp_t[g:g + 1, :])


def _tc_compute(rf, mus, ann_col, conf_col, b):
    n4 = rf.shape[0]
    c, d = mus.shape
    grid = n4 // b
    return pl.pallas_call(
        _tc_body,
        grid=(grid,),
        in_specs=[
            pl.BlockSpec((c, d), lambda i: (0, 0)),
            pl.BlockSpec((b, 4 * d), lambda i: (i, 0)),
            pl.BlockSpec((4 * b, 1), lambda i: (i, 0)),
            pl.BlockSpec((4 * b, 1), lambda i: (i, 0)),
        ],
        out_specs=pl.BlockSpec((c, 4 * b), lambda i: (0, i)),
        out_shape=jax.ShapeDtypeStruct((c, 4 * n4), jnp.float32),
    )(mus, rf, ann_col, conf_col)


def kernel(mus, random_effects, annotators, annotations, confidences):
    n = annotators.shape[0]
    d = random_effects.shape[1]
    rf = _sc_gather(random_effects, annotators, n, d)
    ann_col = annotations.reshape(n, 1)
    conf_col = confidences.reshape(n, 1)
    out_t = _tc_compute(rf, mus, ann_col, conf_col, b=512)
    return out_t.T
